# baseline (speedup 1.0000x reference)
import functools

import jax
import jax.numpy as jnp
from jax import lax
from jax.experimental import pallas as pl
from jax.experimental.pallas import tpu as pltpu
from jax.experimental.pallas import tpu_sc as plsc

_N_NODES = 10000
_N_EDGES = 320000
_D = 192
_N_GRAPHS = 16
_OUT_DIM = 3

_NC = 2
_NS = 16
_NW = _NC * _NS
_CHUNK = 80

_E_SC = 102400
_E_TC = _N_EDGES - _E_SC
_EPW = _E_SC // _NW
_NCHUNK = _EPW // _CHUNK

_BLOCK_E = 6400
_GRID = _E_TC // _BLOCK_E


def _sc_body(eb_hbm, idx_hbm, batch_hbm, out_hbm,
             batch_v, idx_v, g1_v, rows_v, acc_v, sem_in):
    cid = lax.axis_index("c")
    sid = lax.axis_index("s")
    wid = sid * _NC + cid
    base = _E_TC + wid * _EPW

    pltpu.sync_copy(batch_hbm, batch_v)
    pltpu.sync_copy(idx_hbm.at[pl.ds(base, _EPW)], idx_v)

    zeros16 = jnp.zeros((16,), jnp.float32)
    for g in range(_N_GRAPHS):
        for k in range(_D // 16):
            acc_v[g, pl.ds(k * 16, 16)] = zeros16

    def _gather(j, carry):
        iv = idx_v[pl.ds(j * 16, 16)]
        g1_v[pl.ds(j * 16, 16)] = plsc.load_gather(batch_v, [iv])
        return carry

    lax.fori_loop(0, _EPW // 16, _gather, 0)

    iota16 = lax.iota(jnp.int32, 16)
    col_off = [iota16 + k * 16 for k in range(_D // 16)]

    pltpu.async_copy(eb_hbm.at[pl.ds(base, _CHUNK)], rows_v.at[0], sem_in)

    def _chunk(c, carry):
        buf = lax.rem(c, 2)

        @pl.when(c + 1 < _NCHUNK)
        def _():
            pltpu.async_copy(
                eb_hbm.at[pl.ds(base + (c + 1) * _CHUNK, _CHUNK)],
                rows_v.at[1 - buf],
                sem_in,
            )

        pltpu.make_async_copy(
            eb_hbm.at[pl.ds(base + c * _CHUNK, _CHUNK)],
            rows_v.at[buf],
            sem_in,
        ).wait()

        def _edge(e, carry2):
            pos = c * _CHUNK + e
            grow = plsc.load_gather(g1_v, [jnp.full((16,), 0, jnp.int32) + pos])
            for k in range(_D // 16):
                xv = rows_v[buf, e, pl.ds(k * 16, 16)]
                plsc.addupdate_scatter(acc_v, [grow, col_off[k]], xv)
            return carry2

        lax.fori_loop(0, _CHUNK, _edge, 0, unroll=8)
        return carry

    lax.fori_loop(0, _NCHUNK, _chunk, 0)

    pltpu.sync_copy(acc_v, out_hbm.at[wid])


def _partial_sums_sc(edge_embedding, edge_idx, batch):
    mesh = plsc.VectorSubcoreMesh(core_axis_name="c", subcore_axis_name="s")
    return pl.kernel(
        _sc_body,
        mesh=mesh,
        out_type=jax.ShapeDtypeStruct((_NW, _N_GRAPHS, _D), jnp.float32),
        scratch_types=[
            pltpu.VMEM((_N_NODES,), jnp.int32),
            pltpu.VMEM((_EPW,), jnp.int32),
            pltpu.VMEM((_EPW,), jnp.int32),
            pltpu.VMEM((2, _CHUNK, _D), jnp.float32),
            pltpu.VMEM((_N_GRAPHS, _D), jnp.float32),
            pltpu.SemaphoreType.DMA,
        ],
        compiler_params=pltpu.CompilerParams(needs_layout_passes=False),
    )(edge_embedding, edge_idx, batch)


def _tc_body(idx_ref, batch_ref, eb_ref, out_ref, acc_ref, st_ref):
    i = pl.program_id(0)

    @pl.when(i == 0)
    def _():
        bt = batch_ref[...]
        g_iota = jax.lax.broadcasted_iota(jnp.int32, (_N_GRAPHS, _N_NODES), 0)
        st_ref[:, 0:1] = jnp.sum(
            (bt < g_iota).astype(jnp.int32), axis=1, keepdims=True
        )
        st_ref[:, 1:2] = jnp.sum(
            (bt < g_iota + 1).astype(jnp.int32), axis=1, keepdims=True
        )
        acc_ref[...] = jnp.zeros_like(acc_ref)

    starts_lo = st_ref[:, 0:1]
    starts_hi = st_ref[:, 1:2]

    idx = idx_ref[0]
    cmp_lo = (idx >= starts_lo).astype(jnp.float32)
    cmp_hi = (idx >= starts_hi).astype(jnp.float32)
    one_hot = cmp_lo - cmp_hi

    acc_ref[...] += jnp.dot(
        one_hot, eb_ref[...], preferred_element_type=jnp.float32
    )

    @pl.when(i == _GRID - 1)
    def _():
        out_ref[...] = acc_ref[...]


def _partial_sums_tc(edge_embedding, edge_idx, batch2):
    idx3 = edge_idx[:_E_TC].reshape(_GRID, 1, _BLOCK_E)
    return pl.pallas_call(
        _tc_body,
        grid=(_GRID,),
        in_specs=[
            pl.BlockSpec((1, 1, _BLOCK_E), lambda i: (i, 0, 0)),
            pl.BlockSpec((1, _N_NODES), lambda i: (0, 0)),
            pl.BlockSpec((_BLOCK_E, _D), lambda i: (i, 0)),
        ],
        out_specs=pl.BlockSpec((_N_GRAPHS, _D), lambda i: (0, 0)),
        out_shape=jax.ShapeDtypeStruct((_N_GRAPHS, _D), jnp.float32),
        scratch_shapes=[
            pltpu.VMEM((_N_GRAPHS, _D), jnp.float32),
            pltpu.VMEM((_N_GRAPHS, 2), jnp.int32),
        ],
        compiler_params=pltpu.CompilerParams(
            dimension_semantics=("arbitrary",),
        ),
    )(idx3, batch2, edge_embedding)


def _fin_body(tc_ref, parts_ref, batch_ref, W_ref, b_ref, out_ref):
    bt = batch_ref[...]
    g_iota = jax.lax.broadcasted_iota(jnp.int32, (_N_GRAPHS, _N_NODES), 0)
    counts = jnp.sum(
        (bt == g_iota).astype(jnp.float32), axis=1, keepdims=True
    )
    sums = tc_ref[...] + jnp.sum(parts_ref[...], axis=0)
    pooled = sums / jnp.maximum(counts, 1.0)
    out_ref[...] = (
        jnp.dot(pooled, W_ref[...], preferred_element_type=jnp.float32)
        + b_ref[...]
    )


def kernel(edge_embedding, edge_idx, batch, W, b):
    idx32 = edge_idx.astype(jnp.int32)
    batch32 = batch.astype(jnp.int32)
    batch2 = batch32.reshape(1, _N_NODES)
    parts_sc = _partial_sums_sc(edge_embedding, idx32, batch32)
    sums_tc = _partial_sums_tc(edge_embedding, idx32, batch2)
    b2 = b.reshape(1, _OUT_DIM)
    return pl.pallas_call(
        _fin_body,
        grid=(1,),
        in_specs=[
            pl.BlockSpec((_N_GRAPHS, _D), lambda i: (0, 0)),
            pl.BlockSpec((_NW, _N_GRAPHS, _D), lambda i: (0, 0, 0)),
            pl.BlockSpec((1, _N_NODES), lambda i: (0, 0)),
            pl.BlockSpec((_D, _OUT_DIM), lambda i: (0, 0)),
            pl.BlockSpec((1, _OUT_DIM), lambda i: (0, 0)),
        ],
        out_specs=pl.BlockSpec((_N_GRAPHS, _OUT_DIM), lambda i: (0, 0)),
        out_shape=jax.ShapeDtypeStruct((_N_GRAPHS, _OUT_DIM), jnp.float32),
    )(sums_tc, parts_sc, batch2, W, b2)


# speedup vs baseline: 1.1866x; 1.1866x over previous
import jax
import jax.numpy as jnp
from jax.experimental import pallas as pl
from jax.experimental.pallas import tpu as pltpu

_N_NODES = 10000
_N_EDGES = 320000
_D = 192
_N_GRAPHS = 16
_OUT_DIM = 3

_N_STREAMS = 4
_BLOCK_E = 5000
_GRID = _N_EDGES // (_BLOCK_E * _N_STREAMS)


def _body(*refs):
    idx_refs = refs[:_N_STREAMS]
    eb_refs = refs[_N_STREAMS : 2 * _N_STREAMS]
    batch_ref, W_ref, b_ref, out_ref, acc_ref, st_ref = refs[2 * _N_STREAMS :]
    i = pl.program_id(0)

    @pl.when(i == 0)
    def _():
        bt = batch_ref[...]
        g_iota = jax.lax.broadcasted_iota(jnp.int32, (_N_GRAPHS, _N_NODES), 0)
        st_ref[:, 0:1] = jnp.sum(
            (bt < g_iota).astype(jnp.int32), axis=1, keepdims=True
        )
        st_ref[:, 1:2] = jnp.sum(
            (bt < g_iota + 1).astype(jnp.int32), axis=1, keepdims=True
        )
        acc_ref[...] = jnp.zeros_like(acc_ref)

    starts_lo = st_ref[:, 0:1]
    starts_hi = st_ref[:, 1:2]

    partial = jnp.zeros((_N_GRAPHS, _D), dtype=jnp.float32)
    for s in range(_N_STREAMS):
        idx = idx_refs[s][0]
        cmp_lo = (idx >= starts_lo).astype(jnp.float32)
        cmp_hi = (idx >= starts_hi).astype(jnp.float32)
        one_hot = cmp_lo - cmp_hi
        partial += jnp.dot(
            one_hot, eb_refs[s][...], preferred_element_type=jnp.float32
        )
    acc_ref[...] += partial

    @pl.when(i == _GRID - 1)
    def _():
        counts = (starts_hi - starts_lo).astype(jnp.float32)
        pooled = acc_ref[...] / jnp.maximum(counts, 1.0)
        out_ref[...] = (
            jnp.dot(pooled, W_ref[...], preferred_element_type=jnp.float32)
            + b_ref[...]
        )


def kernel(edge_embedding, edge_idx, batch, W, b):
    idx3 = edge_idx.astype(jnp.int32).reshape(-1, 1, _BLOCK_E)
    batch2 = batch.astype(jnp.int32).reshape(1, _N_NODES)
    b2 = b.reshape(1, _OUT_DIM)

    def _idx_spec(s):
        return pl.BlockSpec(
            (1, 1, _BLOCK_E), lambda i, s=s: (i * _N_STREAMS + s, 0, 0)
        )

    def _eb_spec(s):
        return pl.BlockSpec(
            (_BLOCK_E, _D), lambda i, s=s: (i * _N_STREAMS + s, 0)
        )

    return pl.pallas_call(
        _body,
        grid=(_GRID,),
        in_specs=(
            [_idx_spec(s) for s in range(_N_STREAMS)]
            + [_eb_spec(s) for s in range(_N_STREAMS)]
            + [
                pl.BlockSpec((1, _N_NODES), lambda i: (0, 0)),
                pl.BlockSpec((_D, _OUT_DIM), lambda i: (0, 0)),
                pl.BlockSpec((1, _OUT_DIM), lambda i: (0, 0)),
            ]
        ),
        out_specs=pl.BlockSpec((_N_GRAPHS, _OUT_DIM), lambda i: (0, 0)),
        out_shape=jax.ShapeDtypeStruct((_N_GRAPHS, _OUT_DIM), jnp.float32),
        scratch_shapes=[
            pltpu.VMEM((_N_GRAPHS, _D), jnp.float32),
            pltpu.VMEM((_N_GRAPHS, 2), jnp.int32),
        ],
        compiler_params=pltpu.CompilerParams(
            dimension_semantics=("arbitrary",),
        ),
    )(
        *([idx3] * _N_STREAMS),
        *([edge_embedding] * _N_STREAMS),
        batch2,
        W,
        b2,
    )
